# baseline (speedup 1.0000x reference)
import functools

import jax
import jax.numpy as jnp
from jax import lax
from jax.experimental import pallas as pl
from jax.experimental.pallas import tpu as pltpu
from jax.experimental.pallas import tpu_sc as plsc

N = 16384
N_A = 8192
D_A = 64
D_B = 32
D_MODEL = 512
D_FF = 1024
N_ACT = 32
B = 16
BLK = 4096
GRID = N // BLK
A_BLOCKS = N_A // BLK

NC = 2
NS = 16
NW = NC * NS
ROWS_W = N // NW
CH = 128
NCH = ROWS_W // CH

_sc_mesh = functools.partial(
    plsc.VectorSubcoreMesh, core_axis_name="c", subcore_axis_name="s")
_sc_params = pltpu.CompilerParams(use_tc_tiling_on_sc=False)


ZB = 2048
RPT = N // NS


def _scprep_body(idx_hbm, bi_hbm, w16_hbm, e01_hbm,
                 table, idx_v, seg_v, flat_v, ones_v, zbuf, sem):
    cid = lax.axis_index("c")
    sid = lax.axis_index("s")

    for k in range(CH // 16):
        ones_v[pl.ds(k * 16, 16)] = jnp.ones((16,), jnp.float32)

    def _zb(k, c):
        zbuf[pl.ds(k * 16, 16)] = jnp.zeros((16,), jnp.float32)
        return c
    lax.fori_loop(0, ZB // 16, _zb, None)
    for t in range(N // NS * B // ZB):
        pltpu.sync_copy(zbuf, table.at[pl.ds(sid * (N // NS * B) + t * ZB,
                                             ZB)])

    @pl.when(cid == 0)
    def _():
        pltpu.sync_copy(idx_hbm.at[pl.ds(sid * (RPT // CH), RPT // CH)],
                        idx_v)
        cps = [pltpu.async_copy(bi_hbm.at[idx_v.at[r]], seg_v.at[r], sem)
               for r in range(RPT // CH)]
        for cp in cps:
            cp.wait()
        for r in range(RPT // CH):
            for k in range(CH // 16):
                sl = pl.ds(k * 16, 16)
                flat_v[r, sl] = seg_v[r, sl] * N + idx_v[r, sl]
        plsc.subcore_barrier()
        for r in range(RPT // CH):
            pltpu.sync_copy(ones_v, table.at[flat_v.at[r]], add=True)
        plsc.subcore_barrier()
        pltpu.sync_copy(table.at[pl.ds(sid * N, N)],
                        w16_hbm.at[pl.ds(sid * N, N)])

    @pl.when(cid == 1)
    def _():
        for r in range(RPT // CH):
            pltpu.sync_copy(bi_hbm.at[pl.ds(sid * RPT + r * CH, CH)],
                            idx_v.at[r])
        for r in range(RPT // CH):
            for k in range(CH // 16):
                sl = pl.ds(k * 16, 16)
                j0 = sid * RPT + r * CH + k * 16
                flat_v[r, sl] = ((lax.iota(jnp.int32, 16) + j0) * B
                                 + idx_v[r, sl])
        plsc.subcore_barrier()
        for r in range(RPT // CH):
            pltpu.sync_copy(ones_v, table.at[flat_v.at[r]])
        plsc.subcore_barrier()
        pltpu.sync_copy(table.at[pl.ds(sid * N, N)],
                        e01_hbm.at[pl.ds(sid * N, N)])


def _scprep(idx2d, bi):
    k = pl.kernel(
        _scprep_body,
        out_type=(jax.ShapeDtypeStruct((B * N,), jnp.float32),
                  jax.ShapeDtypeStruct((N * B,), jnp.float32)),
        mesh=_sc_mesh(),
        compiler_params=_sc_params,
        scratch_types=[
            pltpu.VMEM_SHARED((B * N,), jnp.float32),
            pltpu.VMEM((RPT // CH, CH), jnp.int32),
            pltpu.VMEM((RPT // CH, CH), jnp.int32),
            pltpu.VMEM((RPT // CH, CH), jnp.int32),
            pltpu.VMEM((CH,), jnp.float32),
            pltpu.VMEM((ZB,), jnp.float32),
            pltpu.SemaphoreType.DMA,
        ],
    )
    return k(idx2d, bi)


def _gather_body(z_hbm, idx_hbm, out_hbm, idx_v, rows_v, sem):
    cid = lax.axis_index("c")
    sid = lax.axis_index("s")
    wid = cid * NS + sid

    pltpu.sync_copy(idx_hbm.at[pl.ds(wid * NCH, NCH)], idx_v)
    cps = []
    for j in range(NCH):
        cps.append(pltpu.async_copy(
            z_hbm.at[idx_v.at[j]], rows_v.at[pl.ds(j * CH, CH)], sem))
    for cp in cps:
        cp.wait()
    pltpu.sync_copy(rows_v, out_hbm.at[pl.ds(wid * ROWS_W, ROWS_W)])


def _gather(z, idx2d):
    k = pl.kernel(
        _gather_body,
        out_type=jax.ShapeDtypeStruct((N, N_ACT), jnp.float32),
        mesh=_sc_mesh(),
        compiler_params=_sc_params,
        scratch_types=[
            pltpu.VMEM((NCH, CH), jnp.int32),
            pltpu.VMEM((ROWS_W, N_ACT), jnp.float32),
            pltpu.SemaphoreType.DMA,
        ],
    )
    return k(z, idx2d)


BLK2 = 1024
GRID2 = N // BLK2
G_ALL = GRID + GRID2


def _tc_body(fa_ref, fb_ref, wa_emb_ref, wb_emb_ref, ba_ref, bb_ref,
             w1_ref, b1_ref, w1t_ref, w2_ref, b2_ref, wh_ref, bh_ref,
             waux_ref, w16_ref, e01_ref, baux_ref,
             z_ref, logits_scr, accs_ref, accc_ref, auxc_ref):
    i = pl.program_id(0)

    @pl.when(i < GRID)
    def _dense():
        is_a = i < A_BLOCKS
        x0f = lax.cond(
            is_a,
            lambda: jnp.dot(fa_ref[...].astype(jnp.bfloat16), wa_emb_ref[...],
                            preferred_element_type=jnp.float32) + ba_ref[...],
            lambda: jnp.dot(fb_ref[...].astype(jnp.bfloat16), wb_emb_ref[...],
                            preferred_element_type=jnp.float32) + bb_ref[...])
        x0 = x0f.astype(jnp.bfloat16)
        b1eff = jnp.where(is_a, b1_ref[...], b1_ref[...] + w1t_ref[...])
        h = jnp.dot(x0, w1_ref[...], preferred_element_type=jnp.float32)
        h = jnp.maximum(h + b1eff, 0.0).astype(jnp.bfloat16)
        yb = (x0.astype(jnp.float32)
              + jnp.dot(h, w2_ref[...], preferred_element_type=jnp.float32)
              + b2_ref[...]).astype(jnp.bfloat16)
        logits_scr[pl.ds(i * BLK, BLK), :] = jnp.dot(
            yb, wh_ref[...], preferred_element_type=jnp.float32) + bh_ref[...]
        v = jnp.dot(yb, waux_ref[...],
                    preferred_element_type=jnp.float32)
        ps = jnp.dot(w16_ref[...], v,
                     preferred_element_type=jnp.float32)
        pc = jnp.sum(w16_ref[...], axis=1, keepdims=True)

        @pl.when(i == 0)
        def _():
            accs_ref[...] = jnp.zeros_like(accs_ref)
            accc_ref[...] = jnp.zeros_like(accc_ref)

        accs_ref[...] += ps
        accc_ref[...] += pc

        @pl.when(i == GRID - 1)
        def _():
            auxc_ref[...] = (accs_ref[...] / jnp.maximum(accc_ref[...], 1.0)
                             + baux_ref[...])

    @pl.when(i >= GRID)
    def _bcast():
        j = i - GRID
        auxm = jnp.dot(e01_ref[...], auxc_ref[...],
                       preferred_element_type=jnp.float32)
        z_ref[...] = logits_scr[pl.ds(j * BLK2, BLK2), :] + auxm


def _tc_dense(fa, fb, wa_emb, wb_emb, ba, bb, w1x, b1r, w1t, w2, b2r,
              wh, bh, waux, w16, e01, baux):
    return pl.pallas_call(
        _tc_body,
        grid=(G_ALL,),
        in_specs=[
            pl.BlockSpec((BLK, D_A), lambda i: (lax.min(i, A_BLOCKS - 1), 0)),
            pl.BlockSpec((BLK, D_B),
                         lambda i: (lax.clamp(0, i - A_BLOCKS,
                                              A_BLOCKS - 1), 0)),
            pl.BlockSpec((D_A, D_MODEL), lambda i: (0, 0)),
            pl.BlockSpec((D_B, D_MODEL), lambda i: (0, 0)),
            pl.BlockSpec((1, D_MODEL), lambda i: (0, 0)),
            pl.BlockSpec((1, D_MODEL), lambda i: (0, 0)),
            pl.BlockSpec((D_MODEL, D_FF), lambda i: (0, 0)),
            pl.BlockSpec((1, D_FF), lambda i: (0, 0)),
            pl.BlockSpec((1, D_FF), lambda i: (0, 0)),
            pl.BlockSpec((D_FF, D_MODEL), lambda i: (0, 0)),
            pl.BlockSpec((1, D_MODEL), lambda i: (0, 0)),
            pl.BlockSpec((D_MODEL, N_ACT), lambda i: (0, 0)),
            pl.BlockSpec((1, N_ACT), lambda i: (0, 0)),
            pl.BlockSpec((D_MODEL, 1), lambda i: (0, 0)),
            pl.BlockSpec((B, BLK), lambda i: (0, lax.min(i, GRID - 1))),
            pl.BlockSpec((BLK2, B),
                         lambda i: (lax.clamp(0, i - GRID, GRID2 - 1), 0)),
            pl.BlockSpec((1, 1), lambda i: (0, 0)),
        ],
        out_specs=pl.BlockSpec((BLK2, N_ACT),
                               lambda i: (lax.clamp(0, i - GRID,
                                                    GRID2 - 1), 0)),
        out_shape=jax.ShapeDtypeStruct((N, N_ACT), jnp.float32),
        scratch_shapes=[
            pltpu.VMEM((N, N_ACT), jnp.float32),
            pltpu.VMEM((B, 1), jnp.float32),
            pltpu.VMEM((B, 1), jnp.float32),
            pltpu.VMEM((B, 1), jnp.float32),
        ],
        compiler_params=pltpu.CompilerParams(
            dimension_semantics=("arbitrary",)),
    )(fa, fb, wa_emb, wb_emb, ba, bb, w1x, b1r, w1t, w2, b2r,
      wh, bh, waux, w16, e01, baux)


def kernel(feats_a, feats_b, batch_index, index_map,
           W_emb_a, b_emb_a, W_emb_b, b_emb_b,
           W1, b1, W2, b2, W_head, b_head, W_aux, b_aux):
    wa_emb = W_emb_a.astype(jnp.bfloat16)
    wb_emb = W_emb_b.astype(jnp.bfloat16)
    ba = b_emb_a.reshape(1, D_MODEL)
    bb = b_emb_b.reshape(1, D_MODEL)
    w1x = W1[:D_MODEL].astype(jnp.bfloat16)
    w1t = W1[D_MODEL].reshape(1, D_FF)
    w2b = W2.astype(jnp.bfloat16)
    whb = W_head.astype(jnp.bfloat16)
    wab = W_aux.astype(jnp.bfloat16)
    b1r = b1.reshape(1, D_FF)
    b2r = b2.reshape(1, D_MODEL)
    bhr = b_head.reshape(1, N_ACT)
    bauxr = b_aux.reshape(1, 1)
    idx2d = index_map.reshape(CH, CH)

    w16f, e01f = _scprep(idx2d, batch_index)
    z = _tc_dense(feats_a, feats_b, wa_emb, wb_emb, ba, bb,
                  w1x, b1r, w1t, w2b, b2r, whb, bhr, wab,
                  w16f.reshape(B, N), e01f.reshape(N, B), bauxr)
    return _gather(z, idx2d)


# speedup vs baseline: 1.0538x; 1.0538x over previous
import functools

import jax
import jax.numpy as jnp
from jax import lax
from jax.experimental import pallas as pl
from jax.experimental.pallas import tpu as pltpu
from jax.experimental.pallas import tpu_sc as plsc

N = 16384
N_A = 8192
D_A = 64
D_B = 32
D_MODEL = 512
D_FF = 1024
N_ACT = 32
B = 16
BLK = 4096
GRID = N // BLK
A_BLOCKS = N_A // BLK

NC = 2
NS = 16
NW = NC * NS
ROWS_W = N // NW
CH = 128
NCH = ROWS_W // CH

_sc_mesh = functools.partial(
    plsc.VectorSubcoreMesh, core_axis_name="c", subcore_axis_name="s")
_sc_params = pltpu.CompilerParams(use_tc_tiling_on_sc=False)


ZB = 2048
RPT = N // NS


def _scprep_body(idx_hbm, bi_hbm, w16_hbm, e01_hbm,
                 table, idx_v, seg_v, flat_v, ones_v, zbuf, sem):
    cid = lax.axis_index("c")
    sid = lax.axis_index("s")

    for k in range(CH // 16):
        ones_v[pl.ds(k * 16, 16)] = jnp.ones((16,), jnp.float32)

    def _zb(k, c):
        zbuf[pl.ds(k * 16, 16)] = jnp.zeros((16,), jnp.float32)
        return c
    lax.fori_loop(0, ZB // 16, _zb, None)
    for t in range(N // NS * B // ZB):
        pltpu.sync_copy(zbuf, table.at[pl.ds(sid * (N // NS * B) + t * ZB,
                                             ZB)])

    @pl.when(cid == 0)
    def _():
        pltpu.sync_copy(idx_hbm.at[pl.ds(sid * (RPT // CH), RPT // CH)],
                        idx_v)
        cps = [pltpu.async_copy(bi_hbm.at[idx_v.at[r]], seg_v.at[r], sem)
               for r in range(RPT // CH)]
        for cp in cps:
            cp.wait()
        for r in range(RPT // CH):
            for k in range(CH // 16):
                sl = pl.ds(k * 16, 16)
                flat_v[r, sl] = seg_v[r, sl] * N + idx_v[r, sl]
        plsc.subcore_barrier()
        for r in range(RPT // CH):
            pltpu.sync_copy(ones_v, table.at[flat_v.at[r]], add=True)
        plsc.subcore_barrier()
        pltpu.sync_copy(table.at[pl.ds(sid * N, N)],
                        w16_hbm.at[pl.ds(sid * N, N)])

    @pl.when(cid == 1)
    def _():
        for r in range(RPT // CH):
            pltpu.sync_copy(bi_hbm.at[pl.ds(sid * RPT + r * CH, CH)],
                            idx_v.at[r])
        for r in range(RPT // CH):
            for k in range(CH // 16):
                sl = pl.ds(k * 16, 16)
                j0 = sid * RPT + r * CH + k * 16
                flat_v[r, sl] = ((lax.iota(jnp.int32, 16) + j0) * B
                                 + idx_v[r, sl])
        plsc.subcore_barrier()
        for r in range(RPT // CH):
            pltpu.sync_copy(ones_v, table.at[flat_v.at[r]])
        plsc.subcore_barrier()
        pltpu.sync_copy(table.at[pl.ds(sid * N, N)],
                        e01_hbm.at[pl.ds(sid * N, N)])


def _scprep(idx2d, bi):
    k = pl.kernel(
        _scprep_body,
        out_type=(jax.ShapeDtypeStruct((B * N,), jnp.float32),
                  jax.ShapeDtypeStruct((N * B,), jnp.float32)),
        mesh=_sc_mesh(),
        compiler_params=_sc_params,
        scratch_types=[
            pltpu.VMEM_SHARED((B * N,), jnp.float32),
            pltpu.VMEM((RPT // CH, CH), jnp.int32),
            pltpu.VMEM((RPT // CH, CH), jnp.int32),
            pltpu.VMEM((RPT // CH, CH), jnp.int32),
            pltpu.VMEM((CH,), jnp.float32),
            pltpu.VMEM((ZB,), jnp.float32),
            pltpu.SemaphoreType.DMA,
        ],
    )
    return k(idx2d, bi)


def _gather_body(z_hbm, idx_hbm, out_hbm, idx_v, rows_v, sem):
    cid = lax.axis_index("c")
    sid = lax.axis_index("s")
    wid = cid * NS + sid

    pltpu.sync_copy(idx_hbm.at[pl.ds(wid * NCH, NCH)], idx_v)
    cps = []
    for j in range(NCH):
        cps.append(pltpu.async_copy(
            z_hbm.at[idx_v.at[j]], rows_v.at[pl.ds(j * CH, CH)], sem))
    for cp in cps:
        cp.wait()
    pltpu.sync_copy(rows_v, out_hbm.at[pl.ds(wid * ROWS_W, ROWS_W)])


def _gather(z, idx2d):
    k = pl.kernel(
        _gather_body,
        out_type=jax.ShapeDtypeStruct((N, N_ACT), jnp.float32),
        mesh=_sc_mesh(),
        compiler_params=_sc_params,
        scratch_types=[
            pltpu.VMEM((NCH, CH), jnp.int32),
            pltpu.VMEM((ROWS_W, N_ACT), jnp.float32),
            pltpu.SemaphoreType.DMA,
        ],
    )
    return k(z, idx2d)


BLK2 = 4096
GRID2 = N // BLK2
G_ALL = GRID + GRID2


def _tc_body(fa_ref, fb_ref, wa_emb_ref, wb_emb_ref, ba_ref, bb_ref,
             w1_ref, b1_ref, w1t_ref, w2_ref, b2_ref, wh_ref, bh_ref,
             waux_ref, w16_ref, e01_ref, baux_ref,
             z_ref, logits_scr, accs_ref, accc_ref, auxc_ref):
    i = pl.program_id(0)

    @pl.when(i < GRID)
    def _dense():
        is_a = i < A_BLOCKS
        x0f = lax.cond(
            is_a,
            lambda: jnp.dot(fa_ref[...].astype(jnp.bfloat16), wa_emb_ref[...],
                            preferred_element_type=jnp.float32) + ba_ref[...],
            lambda: jnp.dot(fb_ref[...].astype(jnp.bfloat16), wb_emb_ref[...],
                            preferred_element_type=jnp.float32) + bb_ref[...])
        x0 = x0f
        b1eff = jnp.where(is_a, b1_ref[...], b1_ref[...] + w1t_ref[...])
        h = jnp.dot(x0.astype(jnp.bfloat16), w1_ref[...],
                    preferred_element_type=jnp.float32)
        h = jnp.maximum(h + b1eff, 0.0).astype(jnp.bfloat16)
        yb = (x0
              + jnp.dot(h, w2_ref[...], preferred_element_type=jnp.float32)
              + b2_ref[...]).astype(jnp.bfloat16)
        logits_scr[pl.ds(i * BLK, BLK), :] = jnp.dot(
            yb, wh_ref[...], preferred_element_type=jnp.float32) + bh_ref[...]
        v = jnp.dot(yb, waux_ref[...],
                    preferred_element_type=jnp.float32)
        ps = jnp.dot(w16_ref[...], v,
                     preferred_element_type=jnp.float32)
        pc = jnp.sum(w16_ref[...], axis=1, keepdims=True)

        @pl.when(i == 0)
        def _():
            accs_ref[...] = jnp.zeros_like(accs_ref)
            accc_ref[...] = jnp.zeros_like(accc_ref)

        accs_ref[...] += ps
        accc_ref[...] += pc

        @pl.when(i == GRID - 1)
        def _():
            auxc_ref[...] = (accs_ref[...] / jnp.maximum(accc_ref[...], 1.0)
                             + baux_ref[...])

    @pl.when(i >= GRID)
    def _bcast():
        j = i - GRID
        auxm = jnp.dot(e01_ref[...], auxc_ref[...],
                       preferred_element_type=jnp.float32)
        z_ref[...] = logits_scr[pl.ds(j * BLK2, BLK2), :] + auxm


def _tc_dense(fa, fb, wa_emb, wb_emb, ba, bb, w1x, b1r, w1t, w2, b2r,
              wh, bh, waux, w16, e01, baux):
    return pl.pallas_call(
        _tc_body,
        grid=(G_ALL,),
        in_specs=[
            pl.BlockSpec((BLK, D_A), lambda i: (lax.min(i, A_BLOCKS - 1), 0)),
            pl.BlockSpec((BLK, D_B),
                         lambda i: (lax.clamp(0, i - A_BLOCKS,
                                              A_BLOCKS - 1), 0)),
            pl.BlockSpec((D_A, D_MODEL), lambda i: (0, 0)),
            pl.BlockSpec((D_B, D_MODEL), lambda i: (0, 0)),
            pl.BlockSpec((1, D_MODEL), lambda i: (0, 0)),
            pl.BlockSpec((1, D_MODEL), lambda i: (0, 0)),
            pl.BlockSpec((D_MODEL, D_FF), lambda i: (0, 0)),
            pl.BlockSpec((1, D_FF), lambda i: (0, 0)),
            pl.BlockSpec((1, D_FF), lambda i: (0, 0)),
            pl.BlockSpec((D_FF, D_MODEL), lambda i: (0, 0)),
            pl.BlockSpec((1, D_MODEL), lambda i: (0, 0)),
            pl.BlockSpec((D_MODEL, N_ACT), lambda i: (0, 0)),
            pl.BlockSpec((1, N_ACT), lambda i: (0, 0)),
            pl.BlockSpec((D_MODEL, 1), lambda i: (0, 0)),
            pl.BlockSpec((B, BLK), lambda i: (0, lax.min(i, GRID - 1))),
            pl.BlockSpec((BLK2, B),
                         lambda i: (lax.clamp(0, i - GRID, GRID2 - 1), 0)),
            pl.BlockSpec((1, 1), lambda i: (0, 0)),
        ],
        out_specs=pl.BlockSpec((BLK2, N_ACT),
                               lambda i: (lax.clamp(0, i - GRID,
                                                    GRID2 - 1), 0)),
        out_shape=jax.ShapeDtypeStruct((N, N_ACT), jnp.float32),
        scratch_shapes=[
            pltpu.VMEM((N, N_ACT), jnp.float32),
            pltpu.VMEM((B, 1), jnp.float32),
            pltpu.VMEM((B, 1), jnp.float32),
            pltpu.VMEM((B, 1), jnp.float32),
        ],
        compiler_params=pltpu.CompilerParams(
            dimension_semantics=("arbitrary",)),
    )(fa, fb, wa_emb, wb_emb, ba, bb, w1x, b1r, w1t, w2, b2r,
      wh, bh, waux, w16, e01, baux)


def kernel(feats_a, feats_b, batch_index, index_map,
           W_emb_a, b_emb_a, W_emb_b, b_emb_b,
           W1, b1, W2, b2, W_head, b_head, W_aux, b_aux):
    wa_emb = W_emb_a.astype(jnp.bfloat16)
    wb_emb = W_emb_b.astype(jnp.bfloat16)
    ba = b_emb_a.reshape(1, D_MODEL)
    bb = b_emb_b.reshape(1, D_MODEL)
    w1x = W1[:D_MODEL].astype(jnp.bfloat16)
    w1t = W1[D_MODEL].reshape(1, D_FF)
    w2b = W2.astype(jnp.bfloat16)
    whb = W_head.astype(jnp.bfloat16)
    wab = W_aux.astype(jnp.bfloat16)
    b1r = b1.reshape(1, D_FF)
    b2r = b2.reshape(1, D_MODEL)
    bhr = b_head.reshape(1, N_ACT)
    bauxr = b_aux.reshape(1, 1)
    idx2d = index_map.reshape(CH, CH)

    w16f, e01f = _scprep(idx2d, batch_index)
    z = _tc_dense(feats_a, feats_b, wa_emb, wb_emb, ba, bb,
                  w1x, b1r, w1t, w2b, b2r, whb, bhr, wab,
                  w16f.reshape(B, N), e01f.reshape(N, B), bauxr)
    return _gather(z, idx2d)
